# Initial kernel scaffold; baseline (speedup 1.0000x reference)
import jax
import jax.numpy as jnp
from jax.experimental import pallas as pl


def kernel(x, edge_index, W_z, b_z, W_r, b_r, W_h, b_h, W_out, b_out):
    raise NotImplementedError("write your pallas kernel here")


# speedup vs baseline: 2.6827x; 2.6827x over previous
import functools

import jax
import jax.numpy as jnp
from jax import lax
from jax.experimental import pallas as pl
from jax.experimental.pallas import tpu as pltpu
from jax.experimental.pallas import tpu_sc as plsc

N_NODES = 10000
HALF = 5000
F = 128
E = 320000
NC, NS = 2, 16
EB = 128
NBLK = 160
EPW = NBLK * EB
E_PAD = NS * EPW
N_ACC = 5120
STRIPE = N_ACC // NS
DUMMY = 5056
RING = 2

def _sc_spmm_body(z_hbm, rows_hbm, cols_hbm, zeros_hbm, out_hbm,
                  idxr, idxc, gbuf, acc,
                  sg0, sg1, sa0, sa1):
    c = lax.axis_index("c")
    s = lax.axis_index("s")
    sems_g = [sg0, sg1]
    sems_a = [sa0, sa1]

    pltpu.sync_copy(rows_hbm.at[s], idxr)
    pltpu.sync_copy(cols_hbm.at[c, s], idxc)
    pltpu.sync_copy(zeros_hbm, acc.at[pl.ds(s * STRIPE, STRIPE)])
    plsc.subcore_barrier()

    for b in range(RING):
        pltpu.async_copy(z_hbm.at[idxr.at[b]], gbuf.at[b], sems_g[b])

    @pl.loop(0, NBLK, step=RING)
    def _(j):
        for b in range(RING):
            jb = j + b
            pltpu.make_async_copy(z_hbm.at[idxr.at[jb]], gbuf.at[b],
                                  sems_g[b]).wait()
            pltpu.async_copy(gbuf.at[b], acc.at[idxc.at[jb]], sems_a[b],
                             add=True)
            nxt = jb + RING

            @pl.when(nxt < NBLK)
            def _():
                pltpu.make_async_copy(gbuf.at[b], acc.at[idxc.at[jb]],
                                      sems_a[b]).wait()
                pltpu.async_copy(z_hbm.at[idxr.at[nxt]], gbuf.at[b],
                                 sems_g[b])

    for b in range(RING):
        jb = NBLK - RING + b
        pltpu.make_async_copy(gbuf.at[b], acc.at[idxc.at[jb]],
                              sems_a[b]).wait()
    plsc.subcore_barrier()
    pltpu.sync_copy(acc.at[pl.ds(s * STRIPE, STRIPE)],
                    out_hbm.at[c, pl.ds(s * STRIPE, STRIPE)])


@functools.lru_cache(maxsize=None)
def _sc_kernels():
    mesh = plsc.VectorSubcoreMesh(core_axis_name="c", subcore_axis_name="s")
    spmm = functools.partial(
        pl.kernel,
        out_type=jax.ShapeDtypeStruct((NC, N_ACC, F), jnp.float32),
        mesh=mesh,
        scratch_types=(
            [pltpu.VMEM((NBLK, EB), jnp.int32),
             pltpu.VMEM((NBLK, EB), jnp.int32),
             pltpu.VMEM((RING, EB, F), jnp.float32),
             pltpu.VMEM_SHARED((N_ACC, F), jnp.float32)]
            + [pltpu.SemaphoreType.DMA] * (2 * RING)),
    )(_sc_spmm_body)
    return spmm


def _sc_spmm(z, rows_g, cols2, zerosF):
    return _sc_kernels()(z, rows_g, cols2, zerosF)


def _assemble(s):
    return jnp.concatenate([s[0, :HALF], s[1, :HALF]], axis=0)


M_BLK = 400
GRID = N_NODES // M_BLK


def _tc_gates_body(x_ref, h_ref, wax_ref, wah_ref, no_ref, ni_ref,
                   p_ref, zz_ref, zr_ref):
    y = (jnp.dot(x_ref[...], wax_ref[...], preferred_element_type=jnp.float32)
         + jnp.dot(h_ref[...], wah_ref[...],
                   preferred_element_type=jnp.float32))
    no = no_ref[...]
    ni = ni_ref[...]
    p_ref[...] = y[:, :256]
    zz_ref[...] = no * y[:, 256:384] + ni * y[:, 384:512]
    zr_ref[...] = no * y[:, 512:640] + ni * y[:, 640:768]


def _tc_mid_body(x_ref, h_ref, p_ref, sz_ref, sr_ref, bz_ref, br_ref,
                 whx_ref, whh_ref, no_ref, ni_ref,
                 z_ref, ph_ref, zh_ref):
    p = p_ref[...]
    z = jax.nn.sigmoid(p[:, :128] + sz_ref[...] + bz_ref[...])
    r = jax.nn.sigmoid(p[:, 128:] + sr_ref[...] + br_ref[...])
    rh = r * h_ref[...]
    yh = (jnp.dot(x_ref[...], whx_ref[...],
                  preferred_element_type=jnp.float32)
          + jnp.dot(rh, whh_ref[...], preferred_element_type=jnp.float32))
    z_ref[...] = z
    ph_ref[...] = yh[:, :128]
    zh_ref[...] = no_ref[...] * yh[:, 128:256] + ni_ref[...] * yh[:, 256:384]


def _tc_out_body(z_ref, h_ref, ph_ref, sh_ref, bh_ref, hn_ref):
    z = z_ref[...]
    ht = jnp.tanh(ph_ref[...] + sh_ref[...] + bh_ref[...])
    hn_ref[...] = z * h_ref[...] + (1.0 - z) * ht


def _tc_out_proj_body(z_ref, h_ref, ph_ref, sh_ref, bh_ref, wo_ref, bo_ref,
                      hn_ref, out_ref):
    z = z_ref[...]
    ht = jnp.tanh(ph_ref[...] + sh_ref[...] + bh_ref[...])
    hn = z * h_ref[...] + (1.0 - z) * ht
    hn_ref[...] = hn
    out_ref[...] = (jnp.dot(hn, wo_ref[...],
                            preferred_element_type=jnp.float32) + bo_ref[...])


def _node_spec(w):
    return pl.BlockSpec((M_BLK, w), lambda m: (m, 0))


def _full_spec(h, w):
    return pl.BlockSpec((h, w), lambda m: (0, 0))


def _tc_gates(xt, h, wax, wah, no, ni):
    return pl.pallas_call(
        _tc_gates_body,
        grid=(GRID,),
        in_specs=[_node_spec(128), _node_spec(128),
                  _full_spec(128, 768), _full_spec(128, 768),
                  _node_spec(1), _node_spec(1)],
        out_specs=[_node_spec(256), _node_spec(128), _node_spec(128)],
        out_shape=[jax.ShapeDtypeStruct((N_NODES, 256), jnp.float32),
                   jax.ShapeDtypeStruct((N_NODES, 128), jnp.float32),
                   jax.ShapeDtypeStruct((N_NODES, 128), jnp.float32)],
    )(xt, h, wax, wah, no, ni)


def _tc_mid(xt, h, p, sz, sr, bz, br, whx, whh, no, ni):
    return pl.pallas_call(
        _tc_mid_body,
        grid=(GRID,),
        in_specs=[_node_spec(128), _node_spec(128), _node_spec(256),
                  _node_spec(128), _node_spec(128),
                  _full_spec(1, 128), _full_spec(1, 128),
                  _full_spec(128, 384), _full_spec(128, 384),
                  _node_spec(1), _node_spec(1)],
        out_specs=[_node_spec(128), _node_spec(128), _node_spec(128)],
        out_shape=[jax.ShapeDtypeStruct((N_NODES, 128), jnp.float32),
                   jax.ShapeDtypeStruct((N_NODES, 128), jnp.float32),
                   jax.ShapeDtypeStruct((N_NODES, 128), jnp.float32)],
    )(xt, h, p, sz, sr, bz, br, whx, whh, no, ni)


def _tc_out(z, h, ph, sh, bh):
    return pl.pallas_call(
        _tc_out_body,
        grid=(GRID,),
        in_specs=[_node_spec(128), _node_spec(128), _node_spec(128),
                  _node_spec(128), _full_spec(1, 128)],
        out_specs=[_node_spec(128)],
        out_shape=[jax.ShapeDtypeStruct((N_NODES, 128), jnp.float32)],
    )(z, h, ph, sh, bh)


def _tc_out_proj(z, h, ph, sh, bh, wo, bo):
    return pl.pallas_call(
        _tc_out_proj_body,
        grid=(GRID,),
        in_specs=[_node_spec(128), _node_spec(128), _node_spec(128),
                  _node_spec(128), _full_spec(1, 128),
                  _full_spec(128, 128), _full_spec(1, 128)],
        out_specs=[_node_spec(128), _node_spec(128)],
        out_shape=[jax.ShapeDtypeStruct((N_NODES, 128), jnp.float32),
                   jax.ShapeDtypeStruct((N_NODES, 128), jnp.float32)],
    )(z, h, ph, sh, bh, wo, bo)


def _core_split(idx):
    parts = []
    for c in range(NC):
        rel = idx - c * HALF
        rel = jnp.where((rel >= 0) & (rel < HALF), rel, DUMMY)
        parts.append(rel.reshape(NS, NBLK, EB))
    return jnp.stack(parts)


def kernel(x, edge_index, W_z, b_z, W_r, b_r, W_h, b_h, W_out, b_out):
    f32 = jnp.float32
    row = edge_index[0]
    col = edge_index[1]
    pad = E_PAD - E
    rows_g = jnp.concatenate(
        [row, jnp.zeros((pad,), jnp.int32)]).reshape(NS, NBLK, EB)
    pad_i = jnp.full((pad,), -1, jnp.int32)
    cols2 = _core_split(jnp.concatenate([col, pad_i]))
    rows2 = _core_split(jnp.concatenate([row, pad_i]))

    zerosF = jnp.zeros((STRIPE, F), f32)
    onesP = jnp.ones((N_NODES, F), f32)

    deg_in = _assemble(_sc_spmm(onesP, rows_g, cols2, zerosF))[:, :1]
    deg_out = _assemble(_sc_spmm(onesP, rows_g, rows2, zerosF))[:, :1]
    no = 1.0 / jnp.maximum(deg_out, 1.0)
    ni = 1.0 / jnp.maximum(deg_in, 1.0)

    wa = jnp.concatenate([W_z[0, 0] + W_z[1, 0], W_r[0, 0] + W_r[1, 0],
                          W_z[0, 1], W_z[1, 1], W_r[0, 1], W_r[1, 1]],
                         axis=1)
    wax, wah = wa[:128], wa[128:]
    wh = jnp.concatenate([W_h[0, 0] + W_h[1, 0], W_h[0, 1], W_h[1, 1]],
                         axis=1)
    whx, whh = wh[:128], wh[128:]
    bz2, br2, bh2, bo2 = (b[None, :] for b in (b_z, b_r, b_h, b_out))

    h_lay = [jnp.zeros((N_NODES, 128), f32) for _ in range(2)]
    outs = []
    for t in range(x.shape[1]):
        layer_in = x[0, t]
        for l in range(2):
            hprev = h_lay[l]
            p, zz, zr = _tc_gates(layer_in, hprev, wax, wah, no, ni)
            sz = _assemble(_sc_spmm(zz, rows_g, cols2, zerosF))
            sr = _assemble(_sc_spmm(zr, rows_g, cols2, zerosF))
            z, ph, zh = _tc_mid(layer_in, hprev, p, sz, sr, bz2, br2,
                                whx, whh, no, ni)
            sh = _assemble(_sc_spmm(zh, rows_g, cols2, zerosF))
            if l == 0:
                hnew = _tc_out(z, hprev, ph, sh, bh2)[0]
            else:
                hnew, out_t = _tc_out_proj(z, hprev, ph, sh, bh2, W_out, bo2)
            h_lay[l] = hnew
            layer_in = hnew
        outs.append(out_t)
    return jnp.stack(outs, axis=0)[None]


# speedup vs baseline: 3.0808x; 1.1484x over previous
import functools

import jax
import jax.numpy as jnp
from jax import lax
from jax.experimental import pallas as pl
from jax.experimental.pallas import tpu as pltpu
from jax.experimental.pallas import tpu_sc as plsc

N_NODES = 10000
HALF = 5000
F = 128
E = 320000
NC, NS = 2, 16
EB = 128
NBLK = 160
RING = 2
NBLK_X = NBLK + 8
EPW = NBLK * EB
EPW_X = NBLK_X * EB
E_PAD = NS * EPW
E_PAD_X = NS * EPW_X
N_ACC = 5120
STRIPE = N_ACC // NS
DUMMY = 5056


def _sc_spmm_body(z_hbm, rows_hbm, cols_hbm, cnt_hbm, zeros_hbm, out_hbm,
                  idxr, idxc, gbuf, acc, nvm,
                  sg0, sg1, sa0, sa1):
    c = lax.axis_index("c")
    s = lax.axis_index("s")
    sems_g = [sg0, sg1]
    sems_a = [sa0, sa1]

    pltpu.sync_copy(rows_hbm.at[c, s], idxr)
    pltpu.sync_copy(cols_hbm.at[c, s], idxc)
    pltpu.sync_copy(cnt_hbm.at[c, s], nvm)
    pltpu.sync_copy(zeros_hbm, acc.at[pl.ds(s * STRIPE, STRIPE)])
    plsc.subcore_barrier()

    nt = nvm[pl.ds(0, 16)][0]

    for b in range(RING):
        pltpu.async_copy(z_hbm.at[idxr.at[b]], gbuf.at[b], sems_g[b])

    @pl.loop(0, NBLK // RING)
    def _(t):
        @pl.when(t < nt)
        def _():
            j = t * RING
            for b in range(RING):
                jb = j + b
                pltpu.make_async_copy(z_hbm.at[idxr.at[jb]], gbuf.at[b],
                                      sems_g[b]).wait()
                pltpu.async_copy(gbuf.at[b], acc.at[idxc.at[jb]], sems_a[b],
                                 add=True)
                pltpu.make_async_copy(gbuf.at[b], acc.at[idxc.at[jb]],
                                      sems_a[b]).wait()
                pltpu.async_copy(z_hbm.at[idxr.at[jb + RING]], gbuf.at[b],
                                 sems_g[b])

    for b in range(RING):
        pltpu.make_async_copy(z_hbm.at[idxr.at[0]], gbuf.at[b],
                              sems_g[b]).wait()
    plsc.subcore_barrier()
    pltpu.sync_copy(acc.at[pl.ds(s * STRIPE, STRIPE)],
                    out_hbm.at[c, pl.ds(s * STRIPE, STRIPE)])


def _sc_count_body(ones_hbm, idx_hbm, zeros_hbm, out_hbm,
                   idxc, onesv, acc):
    c = lax.axis_index("c")
    s = lax.axis_index("s")

    pltpu.sync_copy(idx_hbm.at[c, s], idxc)
    pltpu.sync_copy(ones_hbm, onesv)
    pltpu.sync_copy(zeros_hbm, acc.at[pl.ds(s * STRIPE, STRIPE)])
    plsc.subcore_barrier()

    @pl.loop(0, NBLK)
    def _(j):
        pltpu.sync_copy(onesv, acc.at[idxc.at[j]], add=True)

    plsc.subcore_barrier()
    pltpu.sync_copy(acc.at[pl.ds(s * STRIPE, STRIPE)],
                    out_hbm.at[c, pl.ds(s * STRIPE, STRIPE)])


@functools.lru_cache(maxsize=None)
def _sc_kernels():
    mesh = plsc.VectorSubcoreMesh(core_axis_name="c", subcore_axis_name="s")
    spmm = functools.partial(
        pl.kernel,
        out_type=jax.ShapeDtypeStruct((NC, N_ACC, F), jnp.float32),
        mesh=mesh,
        scratch_types=(
            [pltpu.VMEM((NBLK_X, EB), jnp.int32),
             pltpu.VMEM((NBLK_X, EB), jnp.int32),
             pltpu.VMEM((RING, EB, F), jnp.float32),
             pltpu.VMEM_SHARED((N_ACC, F), jnp.float32),
             pltpu.VMEM((128,), jnp.int32)]
            + [pltpu.SemaphoreType.DMA] * (2 * RING)),
    )(_sc_spmm_body)
    count = functools.partial(
        pl.kernel,
        out_type=jax.ShapeDtypeStruct((NC, N_ACC, F), jnp.float32),
        mesh=mesh,
        scratch_types=(
            [pltpu.VMEM((NBLK, EB), jnp.int32),
             pltpu.VMEM((EB, F), jnp.float32),
             pltpu.VMEM_SHARED((N_ACC, F), jnp.float32)]),
    )(_sc_count_body)
    return spmm, count


def _sc_spmm(z, rows2, cols2, cnt2, zerosF):
    return _sc_kernels()[0](z, rows2, cols2, cnt2, zerosF)


def _sc_count(onesB, idx2, zerosF):
    return _sc_kernels()[1](onesB, idx2, zerosF)


def _assemble(s):
    return jnp.concatenate([s[0, :HALF], s[1, :HALF]], axis=0)


M_BLK = 400
GRID = N_NODES // M_BLK


def _tc_gates_body(x_ref, h_ref, wax_ref, wah_ref, no_ref, ni_ref,
                   p_ref, zz_ref, zr_ref):
    y = (jnp.dot(x_ref[...], wax_ref[...], preferred_element_type=jnp.float32)
         + jnp.dot(h_ref[...], wah_ref[...],
                   preferred_element_type=jnp.float32))
    no = no_ref[...]
    ni = ni_ref[...]
    p_ref[...] = y[:, :256]
    zz_ref[...] = no * y[:, 256:384] + ni * y[:, 384:512]
    zr_ref[...] = no * y[:, 512:640] + ni * y[:, 640:768]


def _tc_mid_body(x_ref, h_ref, p_ref, sz_ref, sr_ref, bz_ref, br_ref,
                 whx_ref, whh_ref, no_ref, ni_ref,
                 z_ref, ph_ref, zh_ref):
    p = p_ref[...]
    z = jax.nn.sigmoid(p[:, :128] + sz_ref[...] + bz_ref[...])
    r = jax.nn.sigmoid(p[:, 128:] + sr_ref[...] + br_ref[...])
    rh = r * h_ref[...]
    yh = (jnp.dot(x_ref[...], whx_ref[...],
                  preferred_element_type=jnp.float32)
          + jnp.dot(rh, whh_ref[...], preferred_element_type=jnp.float32))
    z_ref[...] = z
    ph_ref[...] = yh[:, :128]
    zh_ref[...] = no_ref[...] * yh[:, 128:256] + ni_ref[...] * yh[:, 256:384]


def _tc_out_body(z_ref, h_ref, ph_ref, sh_ref, bh_ref, hn_ref):
    z = z_ref[...]
    ht = jnp.tanh(ph_ref[...] + sh_ref[...] + bh_ref[...])
    hn_ref[...] = z * h_ref[...] + (1.0 - z) * ht


def _tc_out_proj_body(z_ref, h_ref, ph_ref, sh_ref, bh_ref, wo_ref, bo_ref,
                      hn_ref, out_ref):
    z = z_ref[...]
    ht = jnp.tanh(ph_ref[...] + sh_ref[...] + bh_ref[...])
    hn = z * h_ref[...] + (1.0 - z) * ht
    hn_ref[...] = hn
    out_ref[...] = (jnp.dot(hn, wo_ref[...],
                            preferred_element_type=jnp.float32) + bo_ref[...])


def _node_spec(w):
    return pl.BlockSpec((M_BLK, w), lambda m: (m, 0))


def _full_spec(h, w):
    return pl.BlockSpec((h, w), lambda m: (0, 0))


def _tc_gates(xt, h, wax, wah, no, ni):
    return pl.pallas_call(
        _tc_gates_body,
        grid=(GRID,),
        in_specs=[_node_spec(128), _node_spec(128),
                  _full_spec(128, 768), _full_spec(128, 768),
                  _node_spec(1), _node_spec(1)],
        out_specs=[_node_spec(256), _node_spec(128), _node_spec(128)],
        out_shape=[jax.ShapeDtypeStruct((N_NODES, 256), jnp.float32),
                   jax.ShapeDtypeStruct((N_NODES, 128), jnp.float32),
                   jax.ShapeDtypeStruct((N_NODES, 128), jnp.float32)],
    )(xt, h, wax, wah, no, ni)


def _tc_mid(xt, h, p, sz, sr, bz, br, whx, whh, no, ni):
    return pl.pallas_call(
        _tc_mid_body,
        grid=(GRID,),
        in_specs=[_node_spec(128), _node_spec(128), _node_spec(256),
                  _node_spec(128), _node_spec(128),
                  _full_spec(1, 128), _full_spec(1, 128),
                  _full_spec(128, 384), _full_spec(128, 384),
                  _node_spec(1), _node_spec(1)],
        out_specs=[_node_spec(128), _node_spec(128), _node_spec(128)],
        out_shape=[jax.ShapeDtypeStruct((N_NODES, 128), jnp.float32),
                   jax.ShapeDtypeStruct((N_NODES, 128), jnp.float32),
                   jax.ShapeDtypeStruct((N_NODES, 128), jnp.float32)],
    )(xt, h, p, sz, sr, bz, br, whx, whh, no, ni)


def _tc_out(z, h, ph, sh, bh):
    return pl.pallas_call(
        _tc_out_body,
        grid=(GRID,),
        in_specs=[_node_spec(128), _node_spec(128), _node_spec(128),
                  _node_spec(128), _full_spec(1, 128)],
        out_specs=[_node_spec(128)],
        out_shape=[jax.ShapeDtypeStruct((N_NODES, 128), jnp.float32)],
    )(z, h, ph, sh, bh)


def _tc_out_proj(z, h, ph, sh, bh, wo, bo):
    return pl.pallas_call(
        _tc_out_proj_body,
        grid=(GRID,),
        in_specs=[_node_spec(128), _node_spec(128), _node_spec(128),
                  _node_spec(128), _full_spec(1, 128),
                  _full_spec(128, 128), _full_spec(1, 128)],
        out_specs=[_node_spec(128), _node_spec(128)],
        out_shape=[jax.ShapeDtypeStruct((N_NODES, 128), jnp.float32),
                   jax.ShapeDtypeStruct((N_NODES, 128), jnp.float32)],
    )(z, h, ph, sh, bh, wo, bo)


def _interleave(a):
    return a.reshape(EPW_X, NS).T.reshape(NS, NBLK_X, EB)


def _core_split(idx):
    parts = []
    for c in range(NC):
        rel = idx - c * HALF
        rel = jnp.where((rel >= 0) & (rel < HALF), rel, DUMMY)
        parts.append(rel.reshape(NS, NBLK, EB))
    return jnp.stack(parts)


def kernel(x, edge_index, W_z, b_z, W_r, b_r, W_h, b_h, W_out, b_out):
    f32 = jnp.float32
    i32 = jnp.int32
    row = edge_index[0].astype(i32)
    col = edge_index[1].astype(i32)
    pad = E_PAD - E

    mine1 = (col >= HALF).astype(i32)
    perm = jnp.argsort(mine1, stable=True)
    n1 = jnp.sum(mine1)
    n0 = E - n1
    ncs = [n0, n1]
    iota = jnp.arange(E_PAD_X, dtype=i32)
    zpad = jnp.zeros((E_PAD_X - E,), i32)
    rows_parts, cols_parts, trip_parts = [], [], []
    s_ids = jnp.arange(NS, dtype=i32)
    for c in range(NC):
        p = perm if c == 0 else perm[::-1]
        r_c = jnp.concatenate([row[p], zpad])
        c_c = jnp.concatenate([col[p] - c * HALF, zpad])
        valid = iota < ncs[c]
        rows_parts.append(_interleave(jnp.where(valid, r_c, 0)))
        cols_parts.append(_interleave(jnp.where(valid, c_c, DUMMY)))
        cnt_s = jnp.clip((ncs[c] - s_ids + NS - 1) // NS, 0, EPW)
        nb = (cnt_s + EB - 1) // EB
        nb = ((nb + RING - 1) // RING) * RING
        trip_parts.append(jnp.clip(nb, RING, NBLK) // RING)
    rows2 = jnp.stack(rows_parts)
    cols2 = jnp.stack(cols_parts)
    cnt2 = jnp.broadcast_to(
        jnp.stack(trip_parts).reshape(NC, NS, 1),
        (NC, NS, 128))

    pad_i = jnp.full((pad,), -1, i32)
    colsD = _core_split(jnp.concatenate([col, pad_i]))
    rowsD = _core_split(jnp.concatenate([row, pad_i]))

    zerosF = jnp.zeros((STRIPE, F), f32)
    onesB = jnp.ones((EB, F), f32)

    deg_in = _assemble(_sc_count(onesB, colsD, zerosF))[:, :1]
    deg_out = _assemble(_sc_count(onesB, rowsD, zerosF))[:, :1]
    no = 1.0 / jnp.maximum(deg_out, 1.0)
    ni = 1.0 / jnp.maximum(deg_in, 1.0)

    wa = jnp.concatenate([W_z[0, 0] + W_z[1, 0], W_r[0, 0] + W_r[1, 0],
                          W_z[0, 1], W_z[1, 1], W_r[0, 1], W_r[1, 1]],
                         axis=1)
    wax, wah = wa[:128], wa[128:]
    wh = jnp.concatenate([W_h[0, 0] + W_h[1, 0], W_h[0, 1], W_h[1, 1]],
                         axis=1)
    whx, whh = wh[:128], wh[128:]
    bz2, br2, bh2, bo2 = (b[None, :] for b in (b_z, b_r, b_h, b_out))

    h_lay = [jnp.zeros((N_NODES, 128), f32) for _ in range(2)]
    outs = []
    for t in range(x.shape[1]):
        layer_in = x[0, t]
        for l in range(2):
            hprev = h_lay[l]
            p, zz, zr = _tc_gates(layer_in, hprev, wax, wah, no, ni)
            sz = _assemble(_sc_spmm(zz, rows2, cols2, cnt2, zerosF))
            sr = _assemble(_sc_spmm(zr, rows2, cols2, cnt2, zerosF))
            z, ph, zh = _tc_mid(layer_in, hprev, p, sz, sr, bz2, br2,
                                whx, whh, no, ni)
            sh = _assemble(_sc_spmm(zh, rows2, cols2, cnt2, zerosF))
            if l == 0:
                hnew = _tc_out(z, hprev, ph, sh, bh2)[0]
            else:
                hnew, out_t = _tc_out_proj(z, hprev, ph, sh, bh2, W_out, bo2)
            h_lay[l] = hnew
            layer_in = hnew
        outs.append(out_t)
    return jnp.stack(outs, axis=0)[None]
